# baseline (speedup 1.0000x reference)
import functools

import jax
import jax.numpy as jnp
from jax import lax
from jax.experimental import pallas as pl
from jax.experimental.pallas import tpu as pltpu
from jax.experimental.pallas import tpu_sc as plsc

_TN = 512
_TM = 512
_Q = 16
_SC_LANES = 16
_SC_CORES = 2
_SC_SUBCORES = 16
_SC_WORKERS = _SC_CORES * _SC_SUBCORES


def _knn_body(x_ref, yt_ref, idx_ref, w0_ref, w1_ref, w2_ref):
    b = pl.program_id(0)
    x = x_ref[0]
    yt = yt_ref[0]
    S = yt.shape[1]
    x0, x1, x2 = x[:, 0:1], x[:, 1:2], x[:, 2:3]
    y0, y1, y2 = yt[0:1, :], yt[1:2, :], yt[2:3, :]
    e = jnp.dot(x.astype(jnp.bfloat16), yt.astype(jnp.bfloat16),
                preferred_element_type=jnp.float32)
    sx = x0 * x0 + x1 * x1 + x2 * x2
    sy = y0 * y0 + y1 * y1 + y2 * y2
    d = -2.0 * e
    d = d + sx
    d = d + sy
    lane = lax.broadcasted_iota(jnp.int32, d.shape, 1).astype(jnp.float32)
    big = jnp.float32(S)
    idxs, vals = [], []
    for k in range(3):
        m = jnp.min(d, axis=1, keepdims=True)
        i_f = jnp.min(jnp.where(d == m, lane, big), axis=1, keepdims=True)
        if k < 2:
            d = jnp.where(lane == i_f, jnp.inf, d)
        idxs.append(i_f.astype(jnp.int32))
        vals.append(m)
    r0 = 1.0 / (vals[0] + 1e-8)
    r1 = 1.0 / (vals[1] + 1e-8)
    r2 = 1.0 / (vals[2] + 1e-8)
    norm = r0 + r1 + r2
    base = b * S
    idx_ref[0] = jnp.concatenate(
        [idxs[0] + base, idxs[1] + base, idxs[2] + base], axis=1)
    w0_ref[0] = r0 / norm
    w1_ref[0] = r1 / norm
    w2_ref[0] = r2 / norm


def _knn(xyz1, xyz2t):
    B, N, _ = xyz1.shape
    S = xyz2t.shape[2]
    return pl.pallas_call(
        _knn_body,
        grid=(B, N // _TN),
        in_specs=[
            pl.BlockSpec((1, _TN, 3), lambda b, i: (b, i, 0)),
            pl.BlockSpec((1, 3, S), lambda b, i: (b, 0, 0)),
        ],
        out_specs=[
            pl.BlockSpec((1, _TN, 3), lambda b, i: (b, i, 0)),
            pl.BlockSpec((1, _TN, 1), lambda b, i: (b, i, 0)),
            pl.BlockSpec((1, _TN, 1), lambda b, i: (b, i, 0)),
            pl.BlockSpec((1, _TN, 1), lambda b, i: (b, i, 0)),
        ],
        out_shape=[
            jax.ShapeDtypeStruct((B, N, 3), jnp.int32),
            jax.ShapeDtypeStruct((B, N, 1), jnp.float32),
            jax.ShapeDtypeStruct((B, N, 1), jnp.float32),
            jax.ShapeDtypeStruct((B, N, 1), jnp.float32),
        ],
    )(xyz1, xyz2t)


def _interp(feat2f, idxf, w0f, w1f, w2f):
    BN = idxf.shape[0] // 3
    C2 = feat2f.shape[1]
    QW = BN // _SC_WORKERS
    mesh = plsc.VectorSubcoreMesh(core_axis_name="core", subcore_axis_name="subcore")

    @functools.partial(
        pl.kernel,
        mesh=mesh,
        out_type=jax.ShapeDtypeStruct((BN, C2), jnp.float32),
        scratch_types=[
            pltpu.VMEM((3 * QW,), jnp.int32),
            pltpu.VMEM((QW,), jnp.float32),
            pltpu.VMEM((QW,), jnp.float32),
            pltpu.VMEM((QW,), jnp.float32),
            pltpu.VMEM((3 * _Q, C2), jnp.float32),
            pltpu.VMEM((3 * _Q, C2), jnp.float32),
            pltpu.VMEM((_Q, C2), jnp.float32),
            pltpu.VMEM((_Q, C2), jnp.float32),
            pltpu.SemaphoreType.DMA,
            pltpu.SemaphoreType.DMA,
            pltpu.SemaphoreType.DMA,
            pltpu.SemaphoreType.DMA,
        ],
    )
    def k(feat2_hbm, i_hbm, w0_hbm, w1_hbm, w2_hbm, o_hbm,
          idx_v, w0_v, w1_v, w2_v, g_a, g_b, o_a, o_b,
          sem_ga, sem_gb, sem_oa, sem_ob):
        wid = lax.axis_index("subcore") * _SC_CORES + lax.axis_index("core")
        qbase = wid * QW
        nch = QW // _Q
        pltpu.sync_copy(i_hbm.at[pl.ds(3 * qbase, 3 * QW)], idx_v)
        pltpu.sync_copy(w0_hbm.at[pl.ds(qbase, QW)], w0_v)
        pltpu.sync_copy(w1_hbm.at[pl.ds(qbase, QW)], w1_v)
        pltpu.sync_copy(w2_hbm.at[pl.ds(qbase, QW)], w2_v)

        def start_gather(ci, g_ref, sem):
            pltpu.async_copy(
                feat2_hbm.at[idx_v.at[pl.ds(3 * ci * _Q, 3 * _Q)]], g_ref, sem)

        def wait_gather(g_ref, sem):
            pltpu.make_async_copy(feat2_hbm.at[pl.ds(0, 3 * _Q)], g_ref,
                                  sem).wait()

        def wait_store(o_ref, sem):
            pltpu.make_async_copy(o_ref, o_hbm.at[pl.ds(0, _Q)], sem).wait()

        def combine(ci, g_ref, o_ref):
            w0v = w0_v[pl.ds(ci * _Q, _SC_LANES)]
            w1v = w1_v[pl.ds(ci * _Q, _SC_LANES)]
            w2v = w2_v[pl.ds(ci * _Q, _SC_LANES)]
            for j in range(_SC_LANES):
                w0, w1, w2 = w0v[j], w1v[j], w2v[j]

                @pl.loop(0, C2 // (2 * _SC_LANES))
                def _(h):
                    for u in range(2):
                        sl = pl.ds((2 * h + u) * _SC_LANES, _SC_LANES)
                        acc = g_ref[3 * j, sl] * w0
                        acc = acc + g_ref[3 * j + 1, sl] * w1
                        acc = acc + g_ref[3 * j + 2, sl] * w2
                        o_ref[j, sl] = acc

        start_gather(0, g_a, sem_ga)

        @pl.loop(0, nch // 2)
        def _(cj):
            c0 = 2 * cj
            start_gather(c0 + 1, g_b, sem_gb)
            wait_gather(g_a, sem_ga)

            @pl.when(cj > 0)
            def _():
                wait_store(o_a, sem_oa)

            combine(c0, g_a, o_a)
            pltpu.async_copy(o_a, o_hbm.at[pl.ds(qbase + c0 * _Q, _Q)], sem_oa)

            @pl.when(c0 + 2 < nch)
            def _():
                start_gather(c0 + 2, g_a, sem_ga)

            wait_gather(g_b, sem_gb)

            @pl.when(cj > 0)
            def _():
                wait_store(o_b, sem_ob)

            combine(c0 + 1, g_b, o_b)
            pltpu.async_copy(o_b, o_hbm.at[pl.ds(qbase + (c0 + 1) * _Q, _Q)],
                             sem_ob)

        wait_store(o_a, sem_oa)
        wait_store(o_b, sem_ob)

    return k(feat2f, idxf, w0f, w1f, w2f)


def _mm1_body(f1_ref, ip_ref, wa_ref, wb_ref, b1_ref, x1_ref, s_ref, q_ref):
    i = pl.program_id(0)
    x = jnp.dot(f1_ref[...].astype(jnp.bfloat16), wa_ref[...],
                preferred_element_type=jnp.float32)
    x = x + jnp.dot(ip_ref[...].astype(jnp.bfloat16), wb_ref[...],
                    preferred_element_type=jnp.float32)
    x = x + b1_ref[...]
    x1_ref[...] = x.astype(jnp.bfloat16)
    cs = jnp.sum(x, axis=0, keepdims=True)
    cq = jnp.sum(x * x, axis=0, keepdims=True)

    @pl.when(i == 0)
    def _():
        s_ref[...] = cs
        q_ref[...] = cq

    @pl.when(i != 0)
    def _():
        s_ref[...] += cs
        q_ref[...] += cq


def _bn_mm2_body(x1_ref, s_ref, q_ref, g_ref, be_ref, w2_ref, b2_ref,
                 x2_ref, s2_ref, q2_ref, *, inv_n):
    i = pl.program_id(0)
    mean = s_ref[...] * inv_n
    var = q_ref[...] * inv_n - mean * mean
    h = (g_ref[...] * (x1_ref[...].astype(jnp.float32) - mean)
         / jnp.sqrt(var + 1e-5) + be_ref[...])
    h = jnp.maximum(h, 0.0)
    x2 = jnp.dot(h.astype(jnp.bfloat16), w2_ref[...],
                 preferred_element_type=jnp.float32)
    x2 = x2 + b2_ref[...]
    x2_ref[...] = x2.astype(jnp.bfloat16)
    cs = jnp.sum(x2, axis=0, keepdims=True)
    cq = jnp.sum(x2 * x2, axis=0, keepdims=True)

    @pl.when(i == 0)
    def _():
        s2_ref[...] = cs
        q2_ref[...] = cq

    @pl.when(i != 0)
    def _():
        s2_ref[...] += cs
        q2_ref[...] += cq


def _bn_out_body(x2_ref, s_ref, q_ref, g_ref, be_ref, o_ref, *, inv_n):
    mean = s_ref[...] * inv_n
    var = q_ref[...] * inv_n - mean * mean
    h = (g_ref[...] * (x2_ref[...].astype(jnp.float32) - mean)
         / jnp.sqrt(var + 1e-5) + be_ref[...])
    o_ref[...] = jnp.maximum(h, 0.0)


def _row_spec(cols):
    return pl.BlockSpec((_TM, cols), lambda i: (i, 0))


def _full_spec(rows, cols):
    return pl.BlockSpec((rows, cols), lambda i: (0, 0))


def _mlp(f1, interp, W1, b1, gamma1, beta1, W2, b2, gamma2, beta2):
    BN, C1 = f1.shape
    C2 = interp.shape[1]
    H = W1.shape[0]
    inv_n = 1.0 / BN
    w1aT = W1[:, :C1].T.astype(jnp.bfloat16)
    w1bT = W1[:, C1:].T.astype(jnp.bfloat16)
    w2T = W2.T.astype(jnp.bfloat16)
    grid = (BN // _TM,)

    x1, s1, q1 = pl.pallas_call(
        _mm1_body,
        grid=grid,
        in_specs=[
            _row_spec(C1), _row_spec(C2),
            _full_spec(C1, H), _full_spec(C2, H), _full_spec(1, H),
        ],
        out_specs=[_row_spec(H), _full_spec(1, H), _full_spec(1, H)],
        out_shape=[
            jax.ShapeDtypeStruct((BN, H), jnp.bfloat16),
            jax.ShapeDtypeStruct((1, H), jnp.float32),
            jax.ShapeDtypeStruct((1, H), jnp.float32),
        ],
    )(f1, interp, w1aT, w1bT, b1.reshape(1, H))

    x2, s2, q2 = pl.pallas_call(
        functools.partial(_bn_mm2_body, inv_n=inv_n),
        grid=grid,
        in_specs=[
            _row_spec(H), _full_spec(1, H), _full_spec(1, H),
            _full_spec(1, H), _full_spec(1, H),
            _full_spec(H, H), _full_spec(1, H),
        ],
        out_specs=[_row_spec(H), _full_spec(1, H), _full_spec(1, H)],
        out_shape=[
            jax.ShapeDtypeStruct((BN, H), jnp.bfloat16),
            jax.ShapeDtypeStruct((1, H), jnp.float32),
            jax.ShapeDtypeStruct((1, H), jnp.float32),
        ],
    )(x1, s1, q1, gamma1.reshape(1, H), beta1.reshape(1, H), w2T,
      b2.reshape(1, H))

    out = pl.pallas_call(
        functools.partial(_bn_out_body, inv_n=inv_n),
        grid=grid,
        in_specs=[
            _row_spec(H), _full_spec(1, H), _full_spec(1, H),
            _full_spec(1, H), _full_spec(1, H),
        ],
        out_specs=_row_spec(H),
        out_shape=jax.ShapeDtypeStruct((BN, H), jnp.float32),
    )(x2, s2, q2, gamma2.reshape(1, H), beta2.reshape(1, H))
    return out


def kernel(xyz1, feat1, xyz2, feat2, W1, b1, gamma1, beta1, W2, b2, gamma2,
           beta2):
    B, N, _ = xyz1.shape
    S = xyz2.shape[1]
    C1 = feat1.shape[2]
    C2 = feat2.shape[2]
    H = W1.shape[0]
    BN = B * N

    xyz2t = jnp.transpose(xyz2, (0, 2, 1))
    idx, w0, w1, w2 = _knn(xyz1, xyz2t)
    interp = _interp(feat2.reshape(B * S, C2), idx.reshape(BN * 3),
                     w0.reshape(BN), w1.reshape(BN), w2.reshape(BN))
    out = _mlp(feat1.reshape(BN, C1), interp, W1, b1, gamma1, beta1,
               W2, b2, gamma2, beta2)
    return out.reshape(B, N, H)


# speedup vs baseline: 1.2503x; 1.2503x over previous
import functools

import jax
import jax.numpy as jnp
from jax import lax
from jax.experimental import pallas as pl
from jax.experimental.pallas import tpu as pltpu
from jax.experimental.pallas import tpu_sc as plsc

_TN = 512
_TM = 512
_Q = 16
_SC_LANES = 16
_SC_CORES = 2
_SC_SUBCORES = 16
_SC_WORKERS = _SC_CORES * _SC_SUBCORES


def _knn_body(x_ref, yt_ref, idx_ref, w0_ref, w1_ref, w2_ref):
    b = pl.program_id(0)
    x = x_ref[0]
    yt = yt_ref[0]
    S = yt.shape[1]
    x0, x1, x2 = x[:, 0:1], x[:, 1:2], x[:, 2:3]
    y0, y1, y2 = yt[0:1, :], yt[1:2, :], yt[2:3, :]
    e = jnp.dot(x.astype(jnp.bfloat16), yt.astype(jnp.bfloat16),
                preferred_element_type=jnp.float32)
    sx = x0 * x0 + x1 * x1 + x2 * x2
    sy = y0 * y0 + y1 * y1 + y2 * y2
    d = -2.0 * e
    d = d + sx
    d = d + sy
    lane = lax.broadcasted_iota(jnp.int32, d.shape, 1).astype(jnp.float32)
    big = jnp.float32(S)
    idxs, vals = [], []
    for k in range(3):
        m = jnp.min(d, axis=1, keepdims=True)
        i_f = jnp.min(jnp.where(d == m, lane, big), axis=1, keepdims=True)
        if k < 2:
            d = jnp.where(lane == i_f, jnp.inf, d)
        idxs.append(i_f.astype(jnp.int32))
        vals.append(m)
    r0 = 1.0 / (vals[0] + 1e-8)
    r1 = 1.0 / (vals[1] + 1e-8)
    r2 = 1.0 / (vals[2] + 1e-8)
    norm = r0 + r1 + r2
    base = b * S
    idx_ref[0] = jnp.concatenate(
        [idxs[0] + base, idxs[1] + base, idxs[2] + base], axis=1)
    w0_ref[0] = r0 / norm
    w1_ref[0] = r1 / norm
    w2_ref[0] = r2 / norm


def _knn(xyz1, xyz2t):
    B, N, _ = xyz1.shape
    S = xyz2t.shape[2]
    return pl.pallas_call(
        _knn_body,
        grid=(B, N // _TN),
        in_specs=[
            pl.BlockSpec((1, _TN, 3), lambda b, i: (b, i, 0)),
            pl.BlockSpec((1, 3, S), lambda b, i: (b, 0, 0)),
        ],
        out_specs=[
            pl.BlockSpec((1, _TN, 3), lambda b, i: (b, i, 0)),
            pl.BlockSpec((1, _TN, 1), lambda b, i: (b, i, 0)),
            pl.BlockSpec((1, _TN, 1), lambda b, i: (b, i, 0)),
            pl.BlockSpec((1, _TN, 1), lambda b, i: (b, i, 0)),
        ],
        out_shape=[
            jax.ShapeDtypeStruct((B, N, 3), jnp.int32),
            jax.ShapeDtypeStruct((B, N, 1), jnp.float32),
            jax.ShapeDtypeStruct((B, N, 1), jnp.float32),
            jax.ShapeDtypeStruct((B, N, 1), jnp.float32),
        ],
    )(xyz1, xyz2t)


def _interp(feat2f, idxf, w0f, w1f, w2f):
    BN = idxf.shape[0] // 3
    C2 = feat2f.shape[1]
    QW = BN // _SC_WORKERS
    mesh = plsc.VectorSubcoreMesh(core_axis_name="core", subcore_axis_name="subcore")

    @functools.partial(
        pl.kernel,
        mesh=mesh,
        out_type=jax.ShapeDtypeStruct((BN, C2), jnp.float32),
        scratch_types=[
            pltpu.VMEM((3 * QW,), jnp.int32),
            pltpu.VMEM((QW,), jnp.float32),
            pltpu.VMEM((QW,), jnp.float32),
            pltpu.VMEM((QW,), jnp.float32),
            pltpu.VMEM((3 * _Q, C2), jnp.float32),
            pltpu.VMEM((3 * _Q, C2), jnp.float32),
            pltpu.VMEM((_Q, C2), jnp.float32),
            pltpu.VMEM((_Q, C2), jnp.float32),
            pltpu.SemaphoreType.DMA,
            pltpu.SemaphoreType.DMA,
            pltpu.SemaphoreType.DMA,
            pltpu.SemaphoreType.DMA,
        ],
    )
    def k(feat2_hbm, i_hbm, w0_hbm, w1_hbm, w2_hbm, o_hbm,
          idx_v, w0_v, w1_v, w2_v, g_a, g_b, o_a, o_b,
          sem_ga, sem_gb, sem_oa, sem_ob):
        wid = lax.axis_index("subcore") * _SC_CORES + lax.axis_index("core")
        qbase = wid * QW
        nch = QW // _Q
        pltpu.sync_copy(i_hbm.at[pl.ds(3 * qbase, 3 * QW)], idx_v)
        pltpu.sync_copy(w0_hbm.at[pl.ds(qbase, QW)], w0_v)
        pltpu.sync_copy(w1_hbm.at[pl.ds(qbase, QW)], w1_v)
        pltpu.sync_copy(w2_hbm.at[pl.ds(qbase, QW)], w2_v)

        def start_gather(ci, g_ref, sem):
            pltpu.async_copy(
                feat2_hbm.at[idx_v.at[pl.ds(3 * ci * _Q, 3 * _Q)]], g_ref, sem)

        def wait_gather(g_ref, sem):
            pltpu.make_async_copy(feat2_hbm.at[pl.ds(0, 3 * _Q)], g_ref,
                                  sem).wait()

        def wait_store(o_ref, sem):
            pltpu.make_async_copy(o_ref, o_hbm.at[pl.ds(0, _Q)], sem).wait()

        def combine(ci, g_ref, o_ref):
            w0v = w0_v[pl.ds(ci * _Q, _SC_LANES)]
            w1v = w1_v[pl.ds(ci * _Q, _SC_LANES)]
            w2v = w2_v[pl.ds(ci * _Q, _SC_LANES)]
            for j in range(_SC_LANES):
                w0, w1, w2 = w0v[j], w1v[j], w2v[j]

                @pl.loop(0, C2 // _SC_LANES)
                def _(h):
                    sl = pl.ds(h * _SC_LANES, _SC_LANES)
                    acc = g_ref[3 * j, sl] * w0
                    acc = acc + g_ref[3 * j + 1, sl] * w1
                    acc = acc + g_ref[3 * j + 2, sl] * w2
                    o_ref[j, sl] = acc

        start_gather(0, g_a, sem_ga)

        @pl.loop(0, nch // 2)
        def _(cj):
            c0 = 2 * cj
            start_gather(c0 + 1, g_b, sem_gb)
            wait_gather(g_a, sem_ga)

            @pl.when(cj > 0)
            def _():
                wait_store(o_a, sem_oa)

            combine(c0, g_a, o_a)
            pltpu.async_copy(o_a, o_hbm.at[pl.ds(qbase + c0 * _Q, _Q)], sem_oa)

            @pl.when(c0 + 2 < nch)
            def _():
                start_gather(c0 + 2, g_a, sem_ga)

            wait_gather(g_b, sem_gb)

            @pl.when(cj > 0)
            def _():
                wait_store(o_b, sem_ob)

            combine(c0 + 1, g_b, o_b)
            pltpu.async_copy(o_b, o_hbm.at[pl.ds(qbase + (c0 + 1) * _Q, _Q)],
                             sem_ob)

        wait_store(o_a, sem_oa)
        wait_store(o_b, sem_ob)

    return k(feat2f, idxf, w0f, w1f, w2f)


def _mm1_body(f1_ref, ip_ref, wa_ref, wb_ref, b1_ref, x1_ref, s_ref, q_ref):
    i = pl.program_id(0)
    x = jnp.dot(f1_ref[...].astype(jnp.bfloat16), wa_ref[...],
                preferred_element_type=jnp.float32)
    x = x + jnp.dot(ip_ref[...].astype(jnp.bfloat16), wb_ref[...],
                    preferred_element_type=jnp.float32)
    x = x + b1_ref[...]
    x1_ref[...] = x.astype(jnp.bfloat16)
    cs = jnp.sum(x, axis=0, keepdims=True)
    cq = jnp.sum(x * x, axis=0, keepdims=True)

    @pl.when(i == 0)
    def _():
        s_ref[...] = cs
        q_ref[...] = cq

    @pl.when(i != 0)
    def _():
        s_ref[...] += cs
        q_ref[...] += cq


def _bn_mm2_body(x1_ref, s_ref, q_ref, g_ref, be_ref, w2_ref, b2_ref,
                 x2_ref, s2_ref, q2_ref, *, inv_n):
    i = pl.program_id(0)
    mean = s_ref[...] * inv_n
    var = q_ref[...] * inv_n - mean * mean
    h = (g_ref[...] * (x1_ref[...].astype(jnp.float32) - mean)
         / jnp.sqrt(var + 1e-5) + be_ref[...])
    h = jnp.maximum(h, 0.0)
    x2 = jnp.dot(h.astype(jnp.bfloat16), w2_ref[...],
                 preferred_element_type=jnp.float32)
    x2 = x2 + b2_ref[...]
    x2_ref[...] = x2.astype(jnp.bfloat16)
    cs = jnp.sum(x2, axis=0, keepdims=True)
    cq = jnp.sum(x2 * x2, axis=0, keepdims=True)

    @pl.when(i == 0)
    def _():
        s2_ref[...] = cs
        q2_ref[...] = cq

    @pl.when(i != 0)
    def _():
        s2_ref[...] += cs
        q2_ref[...] += cq


def _bn_out_body(x2_ref, s_ref, q_ref, g_ref, be_ref, o_ref, *, inv_n):
    mean = s_ref[...] * inv_n
    var = q_ref[...] * inv_n - mean * mean
    h = (g_ref[...] * (x2_ref[...].astype(jnp.float32) - mean)
         / jnp.sqrt(var + 1e-5) + be_ref[...])
    o_ref[...] = jnp.maximum(h, 0.0)


def _row_spec(cols):
    return pl.BlockSpec((_TM, cols), lambda i: (i, 0))


def _full_spec(rows, cols):
    return pl.BlockSpec((rows, cols), lambda i: (0, 0))


def _mlp(f1, interp, W1, b1, gamma1, beta1, W2, b2, gamma2, beta2):
    BN, C1 = f1.shape
    C2 = interp.shape[1]
    H = W1.shape[0]
    inv_n = 1.0 / BN
    w1aT = W1[:, :C1].T.astype(jnp.bfloat16)
    w1bT = W1[:, C1:].T.astype(jnp.bfloat16)
    w2T = W2.T.astype(jnp.bfloat16)
    grid = (BN // _TM,)

    x1, s1, q1 = pl.pallas_call(
        _mm1_body,
        grid=grid,
        in_specs=[
            _row_spec(C1), _row_spec(C2),
            _full_spec(C1, H), _full_spec(C2, H), _full_spec(1, H),
        ],
        out_specs=[_row_spec(H), _full_spec(1, H), _full_spec(1, H)],
        out_shape=[
            jax.ShapeDtypeStruct((BN, H), jnp.bfloat16),
            jax.ShapeDtypeStruct((1, H), jnp.float32),
            jax.ShapeDtypeStruct((1, H), jnp.float32),
        ],
    )(f1, interp, w1aT, w1bT, b1.reshape(1, H))

    x2, s2, q2 = pl.pallas_call(
        functools.partial(_bn_mm2_body, inv_n=inv_n),
        grid=grid,
        in_specs=[
            _row_spec(H), _full_spec(1, H), _full_spec(1, H),
            _full_spec(1, H), _full_spec(1, H),
            _full_spec(H, H), _full_spec(1, H),
        ],
        out_specs=[_row_spec(H), _full_spec(1, H), _full_spec(1, H)],
        out_shape=[
            jax.ShapeDtypeStruct((BN, H), jnp.bfloat16),
            jax.ShapeDtypeStruct((1, H), jnp.float32),
            jax.ShapeDtypeStruct((1, H), jnp.float32),
        ],
    )(x1, s1, q1, gamma1.reshape(1, H), beta1.reshape(1, H), w2T,
      b2.reshape(1, H))

    out = pl.pallas_call(
        functools.partial(_bn_out_body, inv_n=inv_n),
        grid=grid,
        in_specs=[
            _row_spec(H), _full_spec(1, H), _full_spec(1, H),
            _full_spec(1, H), _full_spec(1, H),
        ],
        out_specs=_row_spec(H),
        out_shape=jax.ShapeDtypeStruct((BN, H), jnp.float32),
    )(x2, s2, q2, gamma2.reshape(1, H), beta2.reshape(1, H))
    return out


def kernel(xyz1, feat1, xyz2, feat2, W1, b1, gamma1, beta1, W2, b2, gamma2,
           beta2):
    B, N, _ = xyz1.shape
    S = xyz2.shape[1]
    C1 = feat1.shape[2]
    C2 = feat2.shape[2]
    H = W1.shape[0]
    BN = B * N

    xyz2t = jnp.transpose(xyz2, (0, 2, 1))
    idx, w0, w1, w2 = _knn(xyz1, xyz2t)
    interp = _interp(feat2.reshape(B * S, C2), idx.reshape(BN * 3),
                     w0.reshape(BN), w1.reshape(BN), w2.reshape(BN))
    out = _mlp(feat1.reshape(BN, C1), interp, W1, b1, gamma1, beta1,
               W2, b2, gamma2, beta2)
    return out.reshape(B, N, H)


# speedup vs baseline: 1.2593x; 1.0072x over previous
import dataclasses
import functools

import jax
import jax.numpy as jnp
from jax import lax
from jax.experimental import pallas as pl
from jax.experimental.pallas import tpu as pltpu
from jax.experimental.pallas import tpu_sc as plsc

_TN = 512
_TM = 512
_Q = 16
_SC_LANES = 16
_SC_CORES = 2
_SC_SUBCORES = 16
_SC_WORKERS = _SC_CORES * _SC_SUBCORES


def _sc_params():
    cp = pltpu.CompilerParams()
    if "needs_layout_passes" in pltpu.CompilerParams.__dataclass_fields__:
        cp = dataclasses.replace(cp, needs_layout_passes=False)
    return cp


def _to_bf16_pair(w):
    u = lax.bitcast_convert_type(w, jnp.uint32)
    hi = (u + jnp.uint32(0x7FFF) + ((u >> 16) & jnp.uint32(1))) >> 16
    word = (hi << 16) | hi
    return lax.bitcast_convert_type(word, jnp.int32)


def _knn_body(x_ref, yt_ref, idx_ref, wc_ref):
    b = pl.program_id(0)
    x = x_ref[0]
    yt = yt_ref[0]
    S = yt.shape[1]
    x0, x1, x2 = x[:, 0:1], x[:, 1:2], x[:, 2:3]
    y0, y1, y2 = yt[0:1, :], yt[1:2, :], yt[2:3, :]
    e = jnp.dot(x.astype(jnp.bfloat16), yt.astype(jnp.bfloat16),
                preferred_element_type=jnp.float32)
    sx = x0 * x0 + x1 * x1 + x2 * x2
    sy = y0 * y0 + y1 * y1 + y2 * y2
    d = -2.0 * e
    d = d + sx
    d = d + sy
    lane = lax.broadcasted_iota(jnp.int32, d.shape, 1).astype(jnp.float32)
    big = jnp.float32(S)
    idxs, vals = [], []
    for k in range(3):
        m = jnp.min(d, axis=1, keepdims=True)
        i_f = jnp.min(jnp.where(d == m, lane, big), axis=1, keepdims=True)
        if k < 2:
            d = jnp.where(lane == i_f, jnp.inf, d)
        idxs.append(i_f.astype(jnp.int32))
        vals.append(m)
    r0 = 1.0 / (vals[0] + 1e-8)
    r1 = 1.0 / (vals[1] + 1e-8)
    r2 = 1.0 / (vals[2] + 1e-8)
    norm = r0 + r1 + r2
    base = b * S
    idx_ref[0] = jnp.concatenate(
        [idxs[0] + base, idxs[1] + base, idxs[2] + base], axis=1)
    TN = r0.shape[0]
    wc_ref[0] = jnp.concatenate(
        [jnp.broadcast_to(_to_bf16_pair(r / norm), (TN, 16))
         for r in (r0, r1, r2)], axis=1)


def _knn(xyz1, xyz2t):
    B, N, _ = xyz1.shape
    S = xyz2t.shape[2]
    return pl.pallas_call(
        _knn_body,
        grid=(B, N // _TN),
        in_specs=[
            pl.BlockSpec((1, _TN, 3), lambda b, i: (b, i, 0)),
            pl.BlockSpec((1, 3, S), lambda b, i: (b, 0, 0)),
        ],
        out_specs=[
            pl.BlockSpec((1, _TN, 3), lambda b, i: (b, i, 0)),
            pl.BlockSpec((1, _TN, 48), lambda b, i: (b, i, 0)),
        ],
        out_shape=[
            jax.ShapeDtypeStruct((B, N, 3), jnp.int32),
            jax.ShapeDtypeStruct((B, N, 48), jnp.int32),
        ],
    )(xyz1, xyz2t)


_PK_ROWS = 16


def _pack_sc(feat2f):
    R, C = feat2f.shape
    RW = R // _SC_WORKERS
    mesh = plsc.VectorSubcoreMesh(core_axis_name="core", subcore_axis_name="subcore")

    @functools.partial(
        pl.kernel,
        mesh=mesh,
        compiler_params=_sc_params(),
        out_type=jax.ShapeDtypeStruct((R, C // 2), jnp.int32),
        scratch_types=[
            pltpu.VMEM((_PK_ROWS, C), jnp.float32),
            pltpu.VMEM((_PK_ROWS, C // 2), jnp.int32),
            pltpu.SemaphoreType.DMA,
        ],
    )
    def pk(x_hbm, o_hbm, xin, xout, sem):
        wid = lax.axis_index("subcore") * _SC_CORES + lax.axis_index("core")
        rbase = wid * RW

        @pl.loop(0, RW // _PK_ROWS)
        def _(ci):
            r0 = rbase + ci * _PK_ROWS
            pltpu.async_copy(x_hbm.at[pl.ds(r0, _PK_ROWS)], xin, sem).wait()
            for r in range(_PK_ROWS):
                @pl.loop(0, C // 32)
                def _(h):
                    a = xin[r, pl.ds(h * 32, 16)]
                    b = xin[r, pl.ds(h * 32 + 16, 16)]
                    p = plsc.pack(a, b, format=plsc.PackFormat.INTERLEAVED)
                    xout[r, pl.ds(h * 16, 16)] = plsc.bitcast(p, jnp.int32)
            pltpu.async_copy(xout, o_hbm.at[pl.ds(r0, _PK_ROWS)], sem).wait()

    return pk(feat2f)


def _interp(f2p, idxf, wcat):
    BN = idxf.shape[0] // 3
    C2i = f2p.shape[1]
    C2 = 2 * C2i
    QW = BN // _SC_WORKERS
    mesh = plsc.VectorSubcoreMesh(core_axis_name="core", subcore_axis_name="subcore")

    @functools.partial(
        pl.kernel,
        mesh=mesh,
        compiler_params=_sc_params(),
        out_type=jax.ShapeDtypeStruct((BN, C2), jnp.float32),
        scratch_types=[
            pltpu.VMEM((3 * QW,), jnp.int32),
            pltpu.VMEM((3 * _Q, C2i), jnp.int32),
            pltpu.VMEM((3 * _Q, C2i), jnp.int32),
            pltpu.VMEM((_Q, 48), jnp.int32),
            pltpu.VMEM((_Q, 48), jnp.int32),
            pltpu.VMEM((_Q, C2), jnp.float32),
            pltpu.VMEM((_Q, C2), jnp.float32),
            pltpu.SemaphoreType.DMA,
            pltpu.SemaphoreType.DMA,
            pltpu.SemaphoreType.DMA,
            pltpu.SemaphoreType.DMA,
            pltpu.SemaphoreType.DMA,
            pltpu.SemaphoreType.DMA,
        ],
    )
    def k(f2_hbm, i_hbm, w_hbm, o_hbm,
          idx_v, g_a, g_b, wv_a, wv_b, o_a, o_b,
          sem_ga, sem_gb, sem_wa, sem_wb, sem_oa, sem_ob):
        wid = lax.axis_index("subcore") * _SC_CORES + lax.axis_index("core")
        qbase = wid * QW
        nch = QW // _Q
        pltpu.sync_copy(i_hbm.at[pl.ds(3 * qbase, 3 * QW)], idx_v)

        def start_chunk(ci, g_ref, wv_ref, sem_g, sem_w):
            pltpu.async_copy(
                f2_hbm.at[idx_v.at[pl.ds(3 * ci * _Q, 3 * _Q)]], g_ref, sem_g)
            pltpu.async_copy(
                w_hbm.at[pl.ds(qbase + ci * _Q, _Q)], wv_ref, sem_w)

        def wait_chunk(g_ref, wv_ref, sem_g, sem_w):
            pltpu.make_async_copy(f2_hbm.at[pl.ds(0, 3 * _Q)], g_ref,
                                  sem_g).wait()
            pltpu.make_async_copy(w_hbm.at[pl.ds(0, _Q)], wv_ref,
                                  sem_w).wait()

        def wait_store(o_ref, sem):
            pltpu.make_async_copy(o_ref, o_hbm.at[pl.ds(0, _Q)], sem).wait()

        def combine(g_ref, wv_ref, o_ref):
            for j in range(_Q):
                w0v = plsc.bitcast(wv_ref[j, pl.ds(0, 16)], jnp.bfloat16)
                w1v = plsc.bitcast(wv_ref[j, pl.ds(16, 16)], jnp.bfloat16)
                w2v = plsc.bitcast(wv_ref[j, pl.ds(32, 16)], jnp.bfloat16)

                @pl.loop(0, C2i // 16)
                def _(h):
                    sl = pl.ds(h * 16, 16)
                    g0 = plsc.bitcast(g_ref[3 * j, sl], jnp.bfloat16)
                    g1 = plsc.bitcast(g_ref[3 * j + 1, sl], jnp.bfloat16)
                    g2 = plsc.bitcast(g_ref[3 * j + 2, sl], jnp.bfloat16)
                    acc = g0 * w0v
                    acc = acc + g1 * w1v
                    acc = acc + g2 * w2v
                    a, c = plsc.unpack(acc, format=plsc.PackFormat.INTERLEAVED)
                    o_ref[j, pl.ds(h * 32, 16)] = a
                    o_ref[j, pl.ds(h * 32 + 16, 16)] = c

        start_chunk(0, g_a, wv_a, sem_ga, sem_wa)

        @pl.loop(0, nch // 2)
        def _(cj):
            c0 = 2 * cj
            start_chunk(c0 + 1, g_b, wv_b, sem_gb, sem_wb)
            wait_chunk(g_a, wv_a, sem_ga, sem_wa)

            @pl.when(cj > 0)
            def _():
                wait_store(o_a, sem_oa)

            combine(g_a, wv_a, o_a)
            pltpu.async_copy(o_a, o_hbm.at[pl.ds(qbase + c0 * _Q, _Q)], sem_oa)

            @pl.when(c0 + 2 < nch)
            def _():
                start_chunk(c0 + 2, g_a, wv_a, sem_ga, sem_wa)

            wait_chunk(g_b, wv_b, sem_gb, sem_wb)

            @pl.when(cj > 0)
            def _():
                wait_store(o_b, sem_ob)

            combine(g_b, wv_b, o_b)
            pltpu.async_copy(o_b, o_hbm.at[pl.ds(qbase + (c0 + 1) * _Q, _Q)],
                             sem_ob)

        wait_store(o_a, sem_oa)
        wait_store(o_b, sem_ob)

    return k(f2p, idxf, wcat)


def _mm1_body(f1_ref, ip_ref, wa_ref, wb_ref, b1_ref, x1_ref, s_ref, q_ref):
    i = pl.program_id(0)
    x = jnp.dot(f1_ref[...].astype(jnp.bfloat16), wa_ref[...],
                preferred_element_type=jnp.float32)
    x = x + jnp.dot(ip_ref[...].astype(jnp.bfloat16), wb_ref[...],
                    preferred_element_type=jnp.float32)
    x = x + b1_ref[...]
    x1_ref[...] = x.astype(jnp.bfloat16)
    cs = jnp.sum(x, axis=0, keepdims=True)
    cq = jnp.sum(x * x, axis=0, keepdims=True)

    @pl.when(i == 0)
    def _():
        s_ref[...] = cs
        q_ref[...] = cq

    @pl.when(i != 0)
    def _():
        s_ref[...] += cs
        q_ref[...] += cq


def _bn_mm2_body(x1_ref, s_ref, q_ref, g_ref, be_ref, w2_ref, b2_ref,
                 x2_ref, s2_ref, q2_ref, *, inv_n):
    i = pl.program_id(0)
    mean = s_ref[...] * inv_n
    var = q_ref[...] * inv_n - mean * mean
    h = (g_ref[...] * (x1_ref[...].astype(jnp.float32) - mean)
         / jnp.sqrt(var + 1e-5) + be_ref[...])
    h = jnp.maximum(h, 0.0)
    x2 = jnp.dot(h.astype(jnp.bfloat16), w2_ref[...],
                 preferred_element_type=jnp.float32)
    x2 = x2 + b2_ref[...]
    x2_ref[...] = x2.astype(jnp.bfloat16)
    cs = jnp.sum(x2, axis=0, keepdims=True)
    cq = jnp.sum(x2 * x2, axis=0, keepdims=True)

    @pl.when(i == 0)
    def _():
        s2_ref[...] = cs
        q2_ref[...] = cq

    @pl.when(i != 0)
    def _():
        s2_ref[...] += cs
        q2_ref[...] += cq


def _bn_out_body(x2_ref, s_ref, q_ref, g_ref, be_ref, o_ref, *, inv_n):
    mean = s_ref[...] * inv_n
    var = q_ref[...] * inv_n - mean * mean
    h = (g_ref[...] * (x2_ref[...].astype(jnp.float32) - mean)
         / jnp.sqrt(var + 1e-5) + be_ref[...])
    o_ref[...] = jnp.maximum(h, 0.0)


def _row_spec(cols):
    return pl.BlockSpec((_TM, cols), lambda i: (i, 0))


def _full_spec(rows, cols):
    return pl.BlockSpec((rows, cols), lambda i: (0, 0))


def _mlp(f1, interp, W1, b1, gamma1, beta1, W2, b2, gamma2, beta2):
    BN, C1 = f1.shape
    C2 = interp.shape[1]
    H = W1.shape[0]
    inv_n = 1.0 / BN
    w1aT = W1[:, :C1].T.astype(jnp.bfloat16)
    w1bT = W1[:, C1:].T.astype(jnp.bfloat16)
    w2T = W2.T.astype(jnp.bfloat16)
    grid = (BN // _TM,)

    x1, s1, q1 = pl.pallas_call(
        _mm1_body,
        grid=grid,
        in_specs=[
            _row_spec(C1), _row_spec(C2),
            _full_spec(C1, H), _full_spec(C2, H), _full_spec(1, H),
        ],
        out_specs=[_row_spec(H), _full_spec(1, H), _full_spec(1, H)],
        out_shape=[
            jax.ShapeDtypeStruct((BN, H), jnp.bfloat16),
            jax.ShapeDtypeStruct((1, H), jnp.float32),
            jax.ShapeDtypeStruct((1, H), jnp.float32),
        ],
    )(f1, interp, w1aT, w1bT, b1.reshape(1, H))

    x2, s2, q2 = pl.pallas_call(
        functools.partial(_bn_mm2_body, inv_n=inv_n),
        grid=grid,
        in_specs=[
            _row_spec(H), _full_spec(1, H), _full_spec(1, H),
            _full_spec(1, H), _full_spec(1, H),
            _full_spec(H, H), _full_spec(1, H),
        ],
        out_specs=[_row_spec(H), _full_spec(1, H), _full_spec(1, H)],
        out_shape=[
            jax.ShapeDtypeStruct((BN, H), jnp.bfloat16),
            jax.ShapeDtypeStruct((1, H), jnp.float32),
            jax.ShapeDtypeStruct((1, H), jnp.float32),
        ],
    )(x1, s1, q1, gamma1.reshape(1, H), beta1.reshape(1, H), w2T,
      b2.reshape(1, H))

    out = pl.pallas_call(
        functools.partial(_bn_out_body, inv_n=inv_n),
        grid=grid,
        in_specs=[
            _row_spec(H), _full_spec(1, H), _full_spec(1, H),
            _full_spec(1, H), _full_spec(1, H),
        ],
        out_specs=_row_spec(H),
        out_shape=jax.ShapeDtypeStruct((BN, H), jnp.float32),
    )(x2, s2, q2, gamma2.reshape(1, H), beta2.reshape(1, H))
    return out


def kernel(xyz1, feat1, xyz2, feat2, W1, b1, gamma1, beta1, W2, b2, gamma2,
           beta2):
    B, N, _ = xyz1.shape
    S = xyz2.shape[1]
    C1 = feat1.shape[2]
    C2 = feat2.shape[2]
    H = W1.shape[0]
    BN = B * N

    xyz2t = jnp.transpose(xyz2, (0, 2, 1))
    idx, wcat = _knn(xyz1, xyz2t)
    f2p = _pack_sc(feat2.reshape(B * S, C2))
    interp = _interp(f2p, idx.reshape(BN * 3), wcat.reshape(BN, 48))
    out = _mlp(feat1.reshape(BN, C1), interp, W1, b1, gamma1, beta1,
               W2, b2, gamma2, beta2)
    return out.reshape(B, N, H)
